# baseline (speedup 1.0000x reference)
import functools

import jax
import jax.numpy as jnp
from jax import lax
from jax.experimental import pallas as pl
from jax.experimental.pallas import tpu as pltpu
from jax.experimental.pallas import tpu_sc as plsc

D = 128
N = 2 * D * D * D
NW = 32
PER_W = N // NW
K = 2048
CH = PER_W // K
T = CH // 2
L = 16
R = 8

_HI_MASK = -65536


def _warp_body(dx_hbm, dy_hbm, dz_hbm, tab_hbm, out_hbm, *sc):
    ddfA, ddfB = sc[0:3], sc[3:6]
    wA, wB = sc[6:9], sc[9:12]
    iA, iB = sc[12:14], sc[14:16]
    gA, gB = sc[16:18], sc[18:20]
    outA, outB = sc[20], sc[21]
    semA, semB, dsemA, dsemB, osemA, osemB = sc[22:28]

    wid = lax.axis_index("s") * 2 + lax.axis_index("c")
    lanes = lax.iota(jnp.int32, L)

    def ddf_copies(c, bufs, sem):
        base = wid * PER_W + c * K
        return [
            pltpu.make_async_copy(dx_hbm.at[pl.ds(base, K)], bufs[0], sem),
            pltpu.make_async_copy(dy_hbm.at[pl.ds(base, K)], bufs[1], sem),
            pltpu.make_async_copy(dz_hbm.at[pl.ds(base, K)], bufs[2], sem),
        ]

    def start_ddf(c, bufs, sem):
        for cp in ddf_copies(c, bufs, sem):
            cp.start()

    def wait_ddf(c, bufs, sem):
        for cp in ddf_copies(c, bufs, sem):
            cp.wait()

    def out_copy(c, buf, sem):
        base = wid * PER_W + c * K
        return pltpu.make_async_copy(buf, out_hbm.at[pl.ds(base, K)], sem)

    def compute_idx(c, ddfv, idxs, ws):
        base = wid * PER_W + c * K
        dxv, dyv, dzv = ddfv
        wxv, wyv, wzv = ws

        def idx_body(i, carry):
            o = i * L
            sl = pl.ds(o, L)
            p = base + o + lanes
            z = p & (D - 1)
            y = (p >> 7) & (D - 1)
            x = (p >> 14) & (D - 1)
            b = p >> 21

            fx = jnp.clip(x.astype(jnp.float32) + dxv[sl], 0.0, float(D - 1))
            fy = jnp.clip(y.astype(jnp.float32) + dyv[sl], 0.0, float(D - 1))
            fz = jnp.clip(z.astype(jnp.float32) + dzv[sl], 0.0, float(D - 1))
            cx = fx.astype(jnp.int32)
            cy = fy.astype(jnp.int32)
            cz = fz.astype(jnp.int32)
            wxv[sl] = fx - cx.astype(jnp.float32)
            wyv[sl] = fy - cy.astype(jnp.float32)
            wzv[sl] = fz - cz.astype(jnp.float32)
            cx1 = jnp.minimum(cx + 1, D - 1)

            del cx1
            idxs[0][sl] = (((((b << 7) + cx) << 7) + cy) << 7) + cz
            return carry

        lax.fori_loop(0, K // L, idx_body, 0)

    def issue_gathers(idxs, gs, sem):
        for j in range(1):
            pltpu.make_async_copy(tab_hbm.at[idxs[j]], gs[j], sem).start()

    def wait_gathers(idxs, gs, sem):
        for j in range(1):
            pltpu.make_async_copy(tab_hbm.at[idxs[j]], gs[j], sem).wait()

    def blend(gs, idxs, ws, outv):
        wxv, wyv, wzv = ws

        def zlerp(v, wz):
            z0 = lax.bitcast_convert_type(v & _HI_MASK, jnp.float32)
            z1 = lax.bitcast_convert_type(v << 16, jnp.float32)
            return z0 + (z1 - z0) * wz

        def blend_body(i, carry):
            o = i * L
            sl = pl.ds(o, L)
            pts = o + lanes
            zero = pts & 0
            wx = wxv[sl]
            wy = wyv[sl]
            wz = wzv[sl]
            v00 = plsc.load_gather(gs[0], [pts, zero])
            v01 = plsc.load_gather(gs[0], [pts, zero + 1])
            v10 = plsc.load_gather(gs[0], [pts, zero + 2])
            v11 = plsc.load_gather(gs[0], [pts, zero + 3])
            a00 = zlerp(v00, wz)
            a01 = zlerp(v01, wz)
            a10 = zlerp(v10, wz)
            a11 = zlerp(v11, wz)
            b0 = a00 + (a01 - a00) * wy
            b1 = a10 + (a11 - a10) * wy
            outv[sl] = b0 + (b1 - b0) * wx
            return carry

        lax.fori_loop(0, K // L, blend_body, 0)

    def body(t, carry):
        c0 = 2 * t
        wait_ddf(c0, ddfA, dsemA)
        start_ddf(c0 + 1, ddfB, dsemB)
        compute_idx(c0, ddfA, iA, wA)
        issue_gathers(iA, gA, semA)

        @pl.when(t > 0)
        def _():
            wait_gathers(iB, gB, semB)

            @pl.when(t > 1)
            def _():
                out_copy(2 * t - 3, outB, osemB).wait()

            blend(gB, iB, wB, outB)
            out_copy(c0 - 1, outB, osemB).start()

        wait_ddf(c0 + 1, ddfB, dsemB)

        @pl.when(t < T - 1)
        def _():
            start_ddf(c0 + 2, ddfA, dsemA)

        compute_idx(c0 + 1, ddfB, iB, wB)
        issue_gathers(iB, gB, semB)

        wait_gathers(iA, gA, semA)

        @pl.when(t > 0)
        def _():
            out_copy(2 * t - 2, outA, osemA).wait()

        blend(gA, iA, wA, outA)
        out_copy(c0, outA, osemA).start()
        return carry

    start_ddf(0, ddfA, dsemA)
    lax.fori_loop(0, T, body, 0)
    wait_gathers(iB, gB, semB)
    out_copy(CH - 3, outB, osemB).wait()
    blend(gB, iB, wB, outB)
    out_copy(CH - 1, outB, osemB).start()
    out_copy(CH - 1, outB, osemB).wait()
    out_copy(CH - 2, outA, osemA).wait()


@functools.partial(jax.jit, static_argnames=())
def _warp(ddf, image):
    dx = ddf[..., 0].reshape(-1)
    dy = ddf[..., 1].reshape(-1)
    dz = ddf[..., 2].reshape(-1)
    img_flat = image.reshape(-1)
    b0 = img_flat.astype(jnp.bfloat16)
    b1 = jnp.concatenate([b0[1:], jnp.zeros((1,), jnp.bfloat16)])
    u0 = lax.bitcast_convert_type(b0, jnp.uint16).astype(jnp.uint32)
    u1 = lax.bitcast_convert_type(b1, jnp.uint16).astype(jnp.uint32)
    zpair = lax.bitcast_convert_type((u0 << 16) | u1, jnp.int32)
    def shift(a, k):
        return jnp.concatenate([a[k:], jnp.zeros((k,), jnp.int32)])

    zeros = jnp.zeros((N,), jnp.int32)
    tab = jnp.stack(
        [zpair, shift(zpair, D), shift(zpair, D * D), shift(zpair, D * D + D),
         zeros, zeros, zeros, zeros], axis=1)

    mesh = plsc.VectorSubcoreMesh(core_axis_name="c", subcore_axis_name="s")
    kern = functools.partial(
        pl.kernel,
        mesh=mesh,
        compiler_params=pltpu.CompilerParams(
            needs_layout_passes=False, use_tc_tiling_on_sc=False),
        out_type=jax.ShapeDtypeStruct((N,), jnp.float32),
        scratch_types=(
            [pltpu.VMEM((K,), jnp.float32) for _ in range(6)]
            + [pltpu.VMEM((K,), jnp.float32) for _ in range(6)]
            + [pltpu.VMEM((K,), jnp.int32) for _ in range(4)]
            + [pltpu.VMEM((K, R), jnp.int32) for _ in range(4)]
            + [pltpu.VMEM((K,), jnp.float32) for _ in range(2)]
            + [pltpu.SemaphoreType.DMA for _ in range(6)]
        ),
    )(_warp_body)
    return kern(dx, dy, dz, tab)


def kernel(ddf, image):
    return _warp(ddf, image).reshape(image.shape)


# speedup vs baseline: 3.2893x; 3.2893x over previous
import functools

import jax
import jax.numpy as jnp
from jax import lax
from jax.experimental import pallas as pl
from jax.experimental.pallas import tpu as pltpu
from jax.experimental.pallas import tpu_sc as plsc

D = 128
N = 2 * D * D * D
NW = 32
PER_W = N // NW
K = 4096
CH = PER_W // K
T = CH // 2
L = 16

_HI_MASK = -65536


def _warp_body(dx_hbm, dy_hbm, dz_hbm, tab_hbm, out_hbm, *sc):
    ddfA, ddfB = sc[0:3], sc[3:6]
    wA, wB = sc[6:9], sc[9:12]
    iA, iB = sc[12:16], sc[16:20]
    gA, gB = sc[20:24], sc[24:28]
    outA, outB = sc[28], sc[29]
    semA, semB, dsemA, dsemB, osemA, osemB = sc[30:36]

    wid = lax.axis_index("s") * 2 + lax.axis_index("c")
    lanes = lax.iota(jnp.int32, L)

    def ddf_copies(c, bufs, sem):
        base = wid * PER_W + c * K
        return [
            pltpu.make_async_copy(dx_hbm.at[pl.ds(base, K)], bufs[0], sem),
            pltpu.make_async_copy(dy_hbm.at[pl.ds(base, K)], bufs[1], sem),
            pltpu.make_async_copy(dz_hbm.at[pl.ds(base, K)], bufs[2], sem),
        ]

    def start_ddf(c, bufs, sem):
        for cp in ddf_copies(c, bufs, sem):
            cp.start()

    def wait_ddf(c, bufs, sem):
        for cp in ddf_copies(c, bufs, sem):
            cp.wait()

    def out_copy(c, buf, sem):
        base = wid * PER_W + c * K
        return pltpu.make_async_copy(buf, out_hbm.at[pl.ds(base, K)], sem)

    def compute_idx(c, ddfv, idxs, ws):
        base = wid * PER_W + c * K
        dxv, dyv, dzv = ddfv
        wxv, wyv, wzv = ws

        def idx_body(i, carry):
            o = i * L
            sl = pl.ds(o, L)
            p = base + o + lanes
            z = p & (D - 1)
            y = (p >> 7) & (D - 1)
            x = (p >> 14) & (D - 1)
            b = p >> 21

            fx = jnp.clip(x.astype(jnp.float32) + dxv[sl], 0.0, float(D - 1))
            fy = jnp.clip(y.astype(jnp.float32) + dyv[sl], 0.0, float(D - 1))
            fz = jnp.clip(z.astype(jnp.float32) + dzv[sl], 0.0, float(D - 1))
            cx = fx.astype(jnp.int32)
            cy = fy.astype(jnp.int32)
            cz = fz.astype(jnp.int32)
            wxv[sl] = fx - cx.astype(jnp.float32)
            wyv[sl] = fy - cy.astype(jnp.float32)
            wzv[sl] = fz - cz.astype(jnp.float32)
            cx1 = jnp.minimum(cx + 1, D - 1)
            cy1 = jnp.minimum(cy + 1, D - 1)

            bx0 = (b << 7) + cx
            bx1 = (b << 7) + cx1
            idxs[0][sl] = (((bx0 << 7) + cy) << 7) + cz
            idxs[1][sl] = (((bx0 << 7) + cy1) << 7) + cz
            idxs[2][sl] = (((bx1 << 7) + cy) << 7) + cz
            idxs[3][sl] = (((bx1 << 7) + cy1) << 7) + cz
            return carry

        lax.fori_loop(0, K // L, idx_body, 0)

    def issue_gathers(idxs, gs, sem):
        for j in range(4):
            pltpu.make_async_copy(tab_hbm.at[idxs[j]], gs[j], sem).start()

    def wait_gathers(idxs, gs, sem):
        for j in range(4):
            pltpu.make_async_copy(tab_hbm.at[idxs[j]], gs[j], sem).wait()

    def blend(gs, ws, outv):
        wxv, wyv, wzv = ws

        def zlerp(v, wz):
            z0 = lax.bitcast_convert_type(v & _HI_MASK, jnp.float32)
            z1 = lax.bitcast_convert_type(v << 16, jnp.float32)
            return z0 + (z1 - z0) * wz

        def blend_body(i, carry):
            sl = pl.ds(i * L, L)
            wx = wxv[sl]
            wy = wyv[sl]
            wz = wzv[sl]
            a00 = zlerp(gs[0][sl], wz)
            a01 = zlerp(gs[1][sl], wz)
            a10 = zlerp(gs[2][sl], wz)
            a11 = zlerp(gs[3][sl], wz)
            b0 = a00 + (a01 - a00) * wy
            b1 = a10 + (a11 - a10) * wy
            outv[sl] = b0 + (b1 - b0) * wx
            return carry

        lax.fori_loop(0, K // L, blend_body, 0)

    def body(t, carry):
        c0 = 2 * t
        wait_ddf(c0, ddfA, dsemA)
        start_ddf(c0 + 1, ddfB, dsemB)
        compute_idx(c0, ddfA, iA, wA)
        issue_gathers(iA, gA, semA)

        @pl.when(t > 0)
        def _():
            wait_gathers(iB, gB, semB)

            @pl.when(t > 1)
            def _():
                out_copy(2 * t - 3, outB, osemB).wait()

            blend(gB, wB, outB)
            out_copy(c0 - 1, outB, osemB).start()

        wait_ddf(c0 + 1, ddfB, dsemB)

        @pl.when(t < T - 1)
        def _():
            start_ddf(c0 + 2, ddfA, dsemA)

        compute_idx(c0 + 1, ddfB, iB, wB)
        issue_gathers(iB, gB, semB)

        wait_gathers(iA, gA, semA)

        @pl.when(t > 0)
        def _():
            out_copy(2 * t - 2, outA, osemA).wait()

        blend(gA, wA, outA)
        out_copy(c0, outA, osemA).start()
        return carry

    start_ddf(0, ddfA, dsemA)
    lax.fori_loop(0, T, body, 0)
    wait_gathers(iB, gB, semB)
    out_copy(CH - 3, outB, osemB).wait()
    blend(gB, wB, outB)
    out_copy(CH - 1, outB, osemB).start()
    out_copy(CH - 1, outB, osemB).wait()
    out_copy(CH - 2, outA, osemA).wait()


@functools.partial(jax.jit, static_argnames=())
def _warp(ddf, image):
    dx = ddf[..., 0].reshape(-1)
    dy = ddf[..., 1].reshape(-1)
    dz = ddf[..., 2].reshape(-1)
    img_flat = image.reshape(-1)
    b0 = img_flat.astype(jnp.bfloat16)
    b1 = jnp.concatenate([b0[1:], jnp.zeros((1,), jnp.bfloat16)])
    u0 = lax.bitcast_convert_type(b0, jnp.uint16).astype(jnp.uint32)
    u1 = lax.bitcast_convert_type(b1, jnp.uint16).astype(jnp.uint32)
    tab = lax.bitcast_convert_type((u0 << 16) | u1, jnp.int32)

    mesh = plsc.VectorSubcoreMesh(core_axis_name="c", subcore_axis_name="s")
    kern = functools.partial(
        pl.kernel,
        mesh=mesh,
        compiler_params=pltpu.CompilerParams(needs_layout_passes=False),
        out_type=jax.ShapeDtypeStruct((N,), jnp.float32),
        scratch_types=(
            [pltpu.VMEM((K,), jnp.float32) for _ in range(6)]
            + [pltpu.VMEM((K,), jnp.float32) for _ in range(6)]
            + [pltpu.VMEM((K,), jnp.int32) for _ in range(8)]
            + [pltpu.VMEM((K,), jnp.int32) for _ in range(8)]
            + [pltpu.VMEM((K,), jnp.float32) for _ in range(2)]
            + [pltpu.SemaphoreType.DMA for _ in range(6)]
        ),
    )(_warp_body)
    return kern(dx, dy, dz, tab)


def kernel(ddf, image):
    return _warp(ddf, image).reshape(image.shape)
